# Initial kernel scaffold; baseline (speedup 1.0000x reference)
import jax
import jax.numpy as jnp
from jax.experimental import pallas as pl


def kernel(edge_index, x):
    raise NotImplementedError("write your pallas kernel here")


# speedup vs baseline: 3.5480x; 3.5480x over previous
import dataclasses
import functools

import jax
import jax.numpy as jnp
from jax import lax
from jax.experimental import pallas as pl
from jax.experimental.pallas import tpu as pltpu
from jax.experimental.pallas import tpu_sc as plsc

EPS = 1e-7
N = 10000
E = 320000
D = 128
NPAD = 10240
NC = 2
NS = 16
NW = NC * NS
EPW = E // NW
CH = 80
NCHUNK = EPW // CH
NPW = NPAD // NW
SENT = E
NB = 8


def _mesh():
    return plsc.VectorSubcoreMesh(core_axis_name="c", subcore_axis_name="s")


def _sc_params():
    return dataclasses.replace(
        pltpu.CompilerParams(), needs_layout_passes=False)


def _phase1(src, dst, x):
    @functools.partial(
        pl.kernel,
        out_type=[
            jax.ShapeDtypeStruct((NW, NPAD), jnp.float32),
            jax.ShapeDtypeStruct((NW, NPAD), jnp.int32),
        ],
        mesh=_mesh(),
        scratch_types=[
            pltpu.VMEM((CH,), jnp.int32),
            pltpu.VMEM((CH,), jnp.int32),
            pltpu.VMEM((CH, D), jnp.float32),
            pltpu.VMEM((CH, D), jnp.float32),
            pltpu.VMEM((NPAD,), jnp.float32),
            pltpu.VMEM((NPAD,), jnp.int32),
            pltpu.SemaphoreType.DMA,
            pltpu.SemaphoreType.DMA,
        ],
        compiler_params=_sc_params(),
    )
    def k(src_hbm, dst_hbm, x_hbm, qout, eout,
          sidx, didx, srows, drows, qtbl, etbl, sem1, sem2):
        wid = lax.axis_index("s") * NC + lax.axis_index("c")
        base = wid * EPW
        lane0 = lax.iota(jnp.int32, 16) == 0

        @pl.loop(0, NPAD, step=16)
        def _(i):
            qtbl[pl.ds(i, 16)] = jnp.full((16,), -1.0, jnp.float32)
            etbl[pl.ds(i, 16)] = jnp.full((16,), SENT, jnp.int32)

        def edge_body(start, e, dnode):
            a = [srows[e, pl.ds(16 * j, 16)] for j in range(NB)]
            b = [drows[e, pl.ds(16 * j, 16)] for j in range(NB)]
            x2v = a[0] * a[0]
            y2v = b[0] * b[0]
            xyv = a[0] * b[0]
            for j in range(1, NB):
                x2v += a[j] * a[j]
                y2v += b[j] * b[j]
                xyv += a[j] * b[j]
            x2 = jnp.sum(x2v)
            y2 = jnp.sum(y2v)
            xy = -jnp.sum(xyv)
            A = 1.0 + 2.0 * xy + y2
            B = 1.0 - x2
            den1 = 1.0 + 2.0 * xy + x2 * y2 + EPS
            sv = None
            for j in range(NB):
                mv = (A * a[j] - B * b[j]) / den1
                sv = mv * mv if sv is None else sv + mv * mv
            q = jnp.sum(sv)
            idxv = jnp.full((16,), dnode, jnp.int32)
            cur = plsc.load_gather(qtbl, [idxv])[0]

            @pl.when(q > cur)
            def _():
                plsc.store_scatter(
                    qtbl, [idxv], jnp.full((16,), q, jnp.float32), mask=lane0)
                plsc.store_scatter(
                    etbl, [idxv], jnp.full((16,), start + e, jnp.int32),
                    mask=lane0)

        @pl.loop(0, NCHUNK)
        def _(c):
            start = base + c * CH
            pltpu.sync_copy(src_hbm.at[pl.ds(start, CH)], sidx)
            pltpu.sync_copy(dst_hbm.at[pl.ds(start, CH)], didx)
            cp1 = pltpu.async_copy(x_hbm.at[sidx], srows, sem1)
            cp2 = pltpu.async_copy(x_hbm.at[didx], drows, sem2)
            cp1.wait()
            cp2.wait()

            @pl.loop(0, CH, step=16)
            def _(g):
                dv = didx[pl.ds(g, 16)]
                for u in range(16):
                    edge_body(start, g + u, dv[u])

        pltpu.sync_copy(qtbl, qout.at[wid])
        pltpu.sync_copy(etbl, eout.at[wid])

    return k(src, dst, x)


def _merge(qt, et):
    def mk(q_ref, e_ref, be_ref):
        bq = q_ref[0:1, :]
        be = e_ref[0:1, :]
        for t in range(1, NW):
            qv = q_ref[t:t + 1, :]
            ev = e_ref[t:t + 1, :]
            take = (qv > bq) | ((qv == bq) & (ev < be))
            bq = jnp.where(take, qv, bq)
            be = jnp.where(take, ev, be)
        be_ref[...] = be

    return pl.pallas_call(
        mk,
        out_shape=jax.ShapeDtypeStruct((1, NPAD), jnp.int32),
    )(qt, et)


def _phase3(best_eid, src, x):
    @functools.partial(
        pl.kernel,
        out_type=jax.ShapeDtypeStruct((NPAD, D), jnp.float32),
        mesh=_mesh(),
        scratch_types=[
            pltpu.VMEM((NPW,), jnp.int32),
            pltpu.VMEM((NPW,), jnp.int32),
            pltpu.VMEM((NPW,), jnp.int32),
            pltpu.VMEM((NPW,), jnp.int32),
            pltpu.VMEM((NPW, D), jnp.float32),
            pltpu.VMEM((NPW, D), jnp.float32),
            pltpu.SemaphoreType.DMA,
        ],
        compiler_params=_sc_params(),
    )
    def k(be_hbm, src_hbm, x_hbm, agg_hbm,
          bidx, msk, sids, nidx, wrows, xrows, sem):
        wid = lax.axis_index("s") * NC + lax.axis_index("c")
        nb = wid * NPW
        pltpu.sync_copy(be_hbm.at[pl.ds(nb, NPW)], bidx)

        lane = lax.iota(jnp.int32, 16)

        @pl.loop(0, NPW, step=16)
        def _(i):
            v = bidx[pl.ds(i, 16)]
            valid = v < SENT
            msk[pl.ds(i, 16)] = valid.astype(jnp.int32)
            bidx[pl.ds(i, 16)] = jnp.where(valid, v, 0)
            nidx[pl.ds(i, 16)] = jnp.minimum(nb + i + lane, N - 1)

        cp1 = pltpu.async_copy(src_hbm.at[bidx], sids, sem)
        cp1.wait()
        cp2 = pltpu.async_copy(x_hbm.at[sids], wrows, sem)
        cp2.wait()
        cp3 = pltpu.async_copy(x_hbm.at[nidx], xrows, sem)
        cp3.wait()

        @pl.loop(0, NPW, step=16)
        def _(g):
            mv16 = msk[pl.ds(g, 16)]
            for u in range(16):
                i = g + u
                w = [wrows[i, pl.ds(16 * j, 16)] for j in range(NB)]
                xn = [xrows[i, pl.ds(16 * j, 16)] for j in range(NB)]
                a2v = w[0] * w[0]
                b2v = xn[0] * xn[0]
                abv = w[0] * xn[0]
                for j in range(1, NB):
                    a2v += w[j] * w[j]
                    b2v += xn[j] * xn[j]
                    abv += w[j] * xn[j]
                a2 = jnp.sum(a2v)
                b2 = jnp.sum(b2v)
                ab = -jnp.sum(abv)
                A = 1.0 + 2.0 * ab + b2
                B = 1.0 - a2
                den1 = 1.0 + 2.0 * ab + a2 * b2 + EPS
                valid = mv16[u] > 0
                for j in range(NB):
                    mv = (A * w[j] - B * xn[j]) / den1
                    wrows[i, pl.ds(16 * j, 16)] = jnp.where(valid, mv, 0.0)

        pltpu.sync_copy(wrows, agg_hbm.at[pl.ds(nb, NPW)])

    return k(best_eid, src, x)


def _update(x, agg):
    def proj(v):
        n = jnp.sqrt(jnp.sum(v * v, axis=-1, keepdims=True) + EPS)
        mx = 1.0 - 1e-5
        sc = jnp.where(n > mx, mx / (n + EPS), 1.0)
        return v * sc

    def uk(x_ref, a_ref, o_ref):
        xv = x_ref[...]
        av = a_ref[...]
        d2 = jnp.sum((xv - av) ** 2, axis=-1, keepdims=True)
        cx = 1.0 - jnp.sum(xv * xv, axis=-1, keepdims=True)
        cy = 1.0 - jnp.sum(av * av, axis=-1, keepdims=True)
        cr = d2 / (cx * cy + EPS)
        xp = proj(xv)
        yp = proj(av)
        d2b = jnp.sum((xp - yp) ** 2, axis=-1, keepdims=True)
        cxb = 1.0 - jnp.sum(xp * xp, axis=-1, keepdims=True)
        cyb = 1.0 - jnp.sum(yp * yp, axis=-1, keepdims=True)
        crn = d2b / (cxb * cyb + EPS)
        factor = jnp.sqrt(jnp.clip(cr / (crn + EPS), 0.25, 4.0))
        ya = proj(yp * factor)
        x2 = jnp.sum(xp * xp, axis=-1, keepdims=True)
        y2 = jnp.sum(ya * ya, axis=-1, keepdims=True)
        xy = jnp.sum(xp * ya, axis=-1, keepdims=True)
        num = (1.0 + 2.0 * xy + y2) * xp + (1.0 - x2) * ya
        den = 1.0 + 2.0 * xy + x2 * y2
        o_ref[...] = num / (den + EPS)

    rows = 1000
    return pl.pallas_call(
        uk,
        grid=(N // rows,),
        in_specs=[
            pl.BlockSpec((rows, D), lambda i: (i, 0)),
            pl.BlockSpec((rows, D), lambda i: (i, 0)),
        ],
        out_specs=pl.BlockSpec((rows, D), lambda i: (i, 0)),
        out_shape=jax.ShapeDtypeStruct((N, D), jnp.float32),
    )(x, agg)


def kernel(edge_index, x):
    src = edge_index[0]
    dst = edge_index[1]
    qt, et = _phase1(src, dst, x)
    be = _merge(qt, et).reshape(NPAD)
    agg = _phase3(be, src, x)
    return _update(x, agg)


# speedup vs baseline: 4.4824x; 1.2634x over previous
import dataclasses
import functools

import jax
import jax.numpy as jnp
from jax import lax
from jax.experimental import pallas as pl
from jax.experimental.pallas import tpu as pltpu
from jax.experimental.pallas import tpu_sc as plsc

EPS = 1e-7
N = 10000
E = 320000
D = 128
NPAD = 10240
NC = 2
NS = 16
NW = NC * NS
EPW = E // NW
CH = 80
NCHUNK = EPW // CH
NPW = NPAD // NW
SENT = E
NB = 8


def _mesh():
    return plsc.VectorSubcoreMesh(core_axis_name="c", subcore_axis_name="s")


def _sc_params():
    return dataclasses.replace(
        pltpu.CompilerParams(), needs_layout_passes=False)


def _phase1(src, dst, x):
    @functools.partial(
        pl.kernel,
        out_type=[
            jax.ShapeDtypeStruct((NW, NPAD), jnp.float32),
            jax.ShapeDtypeStruct((NW, NPAD), jnp.int32),
        ],
        mesh=_mesh(),
        scratch_types=[
            pltpu.VMEM((NCHUNK, CH), jnp.int32),
            pltpu.VMEM((NCHUNK, CH), jnp.int32),
            pltpu.VMEM((CH, D), jnp.float32),
            pltpu.VMEM((CH, D), jnp.float32),
            pltpu.VMEM((CH, D), jnp.float32),
            pltpu.VMEM((CH, D), jnp.float32),
            pltpu.VMEM((NPAD,), jnp.float32),
            pltpu.VMEM((NPAD,), jnp.int32),
            pltpu.SemaphoreType.DMA,
            pltpu.SemaphoreType.DMA,
            pltpu.SemaphoreType.DMA,
            pltpu.SemaphoreType.DMA,
        ],
        compiler_params=_sc_params(),
    )
    def k(src_hbm, dst_hbm, x_hbm, qout, eout,
          sidx, didx, srowsA, drowsA, srowsB, drowsB, qtbl, etbl,
          semA1, semA2, semB1, semB2):
        wid = lax.axis_index("s") * NC + lax.axis_index("c")
        base = wid * EPW
        lane0 = lax.iota(jnp.int32, 16) == 0

        pltpu.sync_copy(src_hbm.at[wid], sidx)
        pltpu.sync_copy(dst_hbm.at[wid], didx)

        @pl.loop(0, NPAD, step=16)
        def _(i):
            qtbl[pl.ds(i, 16)] = jnp.full((16,), -1.0, jnp.float32)
            etbl[pl.ds(i, 16)] = jnp.full((16,), SENT, jnp.int32)

        def prefetch(c, srows, drows, sem1, sem2):
            pltpu.async_copy(x_hbm.at[sidx.at[c]], srows, sem1)
            pltpu.async_copy(x_hbm.at[didx.at[c]], drows, sem2)

        def wait(c, srows, drows, sem1, sem2):
            pltpu.make_async_copy(x_hbm.at[sidx.at[c]], srows, sem1).wait()
            pltpu.make_async_copy(x_hbm.at[didx.at[c]], drows, sem2).wait()

        def edge_body(srows, drows, e, eid, dnode):
            a = [srows[e, pl.ds(16 * j, 16)] for j in range(NB)]
            b = [drows[e, pl.ds(16 * j, 16)] for j in range(NB)]
            x2v = a[0] * a[0]
            y2v = b[0] * b[0]
            xyv = a[0] * b[0]
            for j in range(1, NB):
                x2v += a[j] * a[j]
                y2v += b[j] * b[j]
                xyv += a[j] * b[j]
            x2 = jnp.sum(x2v)
            y2 = jnp.sum(y2v)
            xy = -jnp.sum(xyv)
            A = 1.0 + 2.0 * xy + y2
            B = 1.0 - x2
            den1 = 1.0 + 2.0 * xy + x2 * y2 + EPS
            sv = None
            for j in range(NB):
                mv = (A * a[j] - B * b[j]) / den1
                sv = mv * mv if sv is None else sv + mv * mv
            q = jnp.sum(sv)
            idxv = jnp.full((16,), dnode, jnp.int32)
            cur = plsc.load_gather(qtbl, [idxv])[0]

            @pl.when(q > cur)
            def _():
                plsc.store_scatter(
                    qtbl, [idxv], jnp.full((16,), q, jnp.float32), mask=lane0)
                plsc.store_scatter(
                    etbl, [idxv], jnp.full((16,), eid, jnp.int32),
                    mask=lane0)

        def compute(c, srows, drows):
            @pl.loop(0, CH, step=16)
            def _(g):
                dv = didx[c, pl.ds(g, 16)]
                start = base + c * CH + g
                for u in range(16):
                    edge_body(srows, drows, g + u, start + u, dv[u])

        prefetch(0, srowsA, drowsA, semA1, semA2)

        @pl.loop(0, NCHUNK - 1, step=2)
        def _(c):
            prefetch(c + 1, srowsB, drowsB, semB1, semB2)
            wait(c, srowsA, drowsA, semA1, semA2)
            compute(c, srowsA, drowsA)
            prefetch(c + 2, srowsA, drowsA, semA1, semA2)
            wait(c + 1, srowsB, drowsB, semB1, semB2)
            compute(c + 1, srowsB, drowsB)

        wait(NCHUNK - 1, srowsA, drowsA, semA1, semA2)
        compute(NCHUNK - 1, srowsA, drowsA)

        pltpu.sync_copy(qtbl, qout.at[wid])
        pltpu.sync_copy(etbl, eout.at[wid])

    return k(src, dst, x)


def _merge(qt, et):
    def mk(q_ref, e_ref, be_ref):
        bq = q_ref[0:1, :]
        be = e_ref[0:1, :]
        for t in range(1, NW):
            qv = q_ref[t:t + 1, :]
            ev = e_ref[t:t + 1, :]
            take = (qv > bq) | ((qv == bq) & (ev < be))
            bq = jnp.where(take, qv, bq)
            be = jnp.where(take, ev, be)
        be_ref[...] = be

    return pl.pallas_call(
        mk,
        out_shape=jax.ShapeDtypeStruct((1, NPAD), jnp.int32),
    )(qt, et)


def _phase3(best_eid, src, x):
    @functools.partial(
        pl.kernel,
        out_type=jax.ShapeDtypeStruct((NPAD, D), jnp.float32),
        mesh=_mesh(),
        scratch_types=[
            pltpu.VMEM((NPW,), jnp.int32),
            pltpu.VMEM((NPW,), jnp.int32),
            pltpu.VMEM((NPW,), jnp.int32),
            pltpu.VMEM((NPW,), jnp.int32),
            pltpu.VMEM((NPW, D), jnp.float32),
            pltpu.VMEM((NPW, D), jnp.float32),
            pltpu.SemaphoreType.DMA,
        ],
        compiler_params=_sc_params(),
    )
    def k(be_hbm, src_hbm, x_hbm, agg_hbm,
          bidx, msk, sids, nidx, wrows, xrows, sem):
        wid = lax.axis_index("s") * NC + lax.axis_index("c")
        nb = wid * NPW
        pltpu.sync_copy(be_hbm.at[pl.ds(nb, NPW)], bidx)

        lane = lax.iota(jnp.int32, 16)

        @pl.loop(0, NPW, step=16)
        def _(i):
            v = bidx[pl.ds(i, 16)]
            valid = v < SENT
            msk[pl.ds(i, 16)] = valid.astype(jnp.int32)
            bidx[pl.ds(i, 16)] = jnp.where(valid, v, 0)
            nidx[pl.ds(i, 16)] = jnp.minimum(nb + i + lane, N - 1)

        cp1 = pltpu.async_copy(src_hbm.at[bidx], sids, sem)
        cp1.wait()
        cp2 = pltpu.async_copy(x_hbm.at[sids], wrows, sem)
        cp2.wait()
        cp3 = pltpu.async_copy(x_hbm.at[nidx], xrows, sem)
        cp3.wait()

        @pl.loop(0, NPW, step=16)
        def _(g):
            mv16 = msk[pl.ds(g, 16)]
            for u in range(16):
                i = g + u
                w = [wrows[i, pl.ds(16 * j, 16)] for j in range(NB)]
                xn = [xrows[i, pl.ds(16 * j, 16)] for j in range(NB)]
                a2v = w[0] * w[0]
                b2v = xn[0] * xn[0]
                abv = w[0] * xn[0]
                for j in range(1, NB):
                    a2v += w[j] * w[j]
                    b2v += xn[j] * xn[j]
                    abv += w[j] * xn[j]
                a2 = jnp.sum(a2v)
                b2 = jnp.sum(b2v)
                ab = -jnp.sum(abv)
                A = 1.0 + 2.0 * ab + b2
                B = 1.0 - a2
                den1 = 1.0 + 2.0 * ab + a2 * b2 + EPS
                valid = mv16[u] > 0
                for j in range(NB):
                    mv = (A * w[j] - B * xn[j]) / den1
                    wrows[i, pl.ds(16 * j, 16)] = jnp.where(valid, mv, 0.0)

        pltpu.sync_copy(wrows, agg_hbm.at[pl.ds(nb, NPW)])

    return k(best_eid, src, x)


def _update(x, agg):
    def proj(v):
        n = jnp.sqrt(jnp.sum(v * v, axis=-1, keepdims=True) + EPS)
        mx = 1.0 - 1e-5
        sc = jnp.where(n > mx, mx / (n + EPS), 1.0)
        return v * sc

    def uk(x_ref, a_ref, o_ref):
        xv = x_ref[...]
        av = a_ref[...]
        d2 = jnp.sum((xv - av) ** 2, axis=-1, keepdims=True)
        cx = 1.0 - jnp.sum(xv * xv, axis=-1, keepdims=True)
        cy = 1.0 - jnp.sum(av * av, axis=-1, keepdims=True)
        cr = d2 / (cx * cy + EPS)
        xp = proj(xv)
        yp = proj(av)
        d2b = jnp.sum((xp - yp) ** 2, axis=-1, keepdims=True)
        cxb = 1.0 - jnp.sum(xp * xp, axis=-1, keepdims=True)
        cyb = 1.0 - jnp.sum(yp * yp, axis=-1, keepdims=True)
        crn = d2b / (cxb * cyb + EPS)
        factor = jnp.sqrt(jnp.clip(cr / (crn + EPS), 0.25, 4.0))
        ya = proj(yp * factor)
        x2 = jnp.sum(xp * xp, axis=-1, keepdims=True)
        y2 = jnp.sum(ya * ya, axis=-1, keepdims=True)
        xy = jnp.sum(xp * ya, axis=-1, keepdims=True)
        num = (1.0 + 2.0 * xy + y2) * xp + (1.0 - x2) * ya
        den = 1.0 + 2.0 * xy + x2 * y2
        o_ref[...] = num / (den + EPS)

    rows = 1000
    return pl.pallas_call(
        uk,
        grid=(N // rows,),
        in_specs=[
            pl.BlockSpec((rows, D), lambda i: (i, 0)),
            pl.BlockSpec((rows, D), lambda i: (i, 0)),
        ],
        out_specs=pl.BlockSpec((rows, D), lambda i: (i, 0)),
        out_shape=jax.ShapeDtypeStruct((N, D), jnp.float32),
    )(x, agg)


def kernel(edge_index, x):
    src = edge_index[0]
    dst = edge_index[1]
    qt, et = _phase1(src.reshape(NW, NCHUNK, CH),
                     dst.reshape(NW, NCHUNK, CH), x)
    be = _merge(qt, et).reshape(NPAD)
    agg = _phase3(be, src, x)
    return _update(x, agg)
